# baseline (speedup 1.0000x reference)
import functools

import jax
import jax.numpy as jnp
from jax import lax
from jax.experimental import pallas as pl
from jax.experimental.pallas import tpu as pltpu
from jax.experimental.pallas import tpu_sc as plsc

N_NODES = 10000
N_EDGES = 320000
D = 128
DH = D // 2

NC = 2
NS = 16
E_PER_T = N_EDGES // NS
C = 80
SEG = 4000
NSEG = E_PER_T // SEG
CPS = SEG // C
NBUF = 5
RCHUNK = 80
N_RCHUNK = N_NODES // RCHUNK


def _mm_body(x_ref, w_ref, oa_ref, ob_ref):
    h = jnp.dot(x_ref[...], w_ref[...], preferred_element_type=jnp.float32)
    oa_ref[...] = h[:, :DH]
    ob_ref[...] = h[:, DH:]


def _matmul(x, W):
    return pl.pallas_call(
        _mm_body,
        grid=(5,),
        in_specs=[
            pl.BlockSpec((2000, D), lambda i: (i, 0)),
            pl.BlockSpec((D, D), lambda i: (0, 0)),
        ],
        out_specs=[
            pl.BlockSpec((2000, DH), lambda i: (i, 0)),
            pl.BlockSpec((2000, DH), lambda i: (i, 0)),
        ],
        out_shape=[
            jax.ShapeDtypeStruct((N_NODES, DH), jnp.float32),
            jax.ShapeDtypeStruct((N_NODES, DH), jnp.float32),
        ],
    )(x, W)


def _sc_aggregate(edges, ha, hb, b):
    mesh = plsc.VectorSubcoreMesh(core_axis_name="c", subcore_axis_name="s")

    @functools.partial(
        pl.kernel,
        out_type=jax.ShapeDtypeStruct((2 * N_NODES, DH), jnp.float32),
        mesh=mesh,
        compiler_params=pltpu.CompilerParams(use_tc_tiling_on_sc=False),
        scratch_types=[
            pltpu.VMEM((2, SEG), jnp.int32),
            pltpu.VMEM((2, SEG), jnp.int32),
            pltpu.VMEM((C,), jnp.int32),
            pltpu.VMEM((NBUF, C, DH), jnp.float32),
            pltpu.VMEM((RCHUNK, DH), jnp.float32),
            pltpu.VMEM((RCHUNK,), jnp.int32),
            pltpu.VMEM((DH,), jnp.float32),
            pltpu.VMEM_SHARED((N_NODES, DH), jnp.float32),
            pltpu.VMEM_SHARED((N_NODES, DH), jnp.float32),
            [pltpu.SemaphoreType.DMA] * NBUF,
            [pltpu.SemaphoreType.DMA] * 2,
        ],
    )
    def body(edges_hbm, ha_hbm, hb_hbm, b_hbm, out_hbm, src_seg, dst_seg,
             dst_cur, msg, robuf, ridx, bvec, agg, h_buf, sems, rsems):
        c = lax.axis_index("c")
        s = lax.axis_index("s")
        ebase = pl.multiple_of(s * E_PER_T, 8)

        def refill(m):
            sb = m % 2
            off = pl.multiple_of(ebase + m * SEG, 8)
            pltpu.async_copy(edges_hbm.at[pl.ds(off, SEG)],
                             src_seg.at[sb], rsems[0])
            pltpu.async_copy(edges_hbm.at[pl.ds(N_EDGES + off, SEG)],
                             dst_seg.at[sb], rsems[1])

        def refill_wait(m):
            sb = m % 2
            off = pl.multiple_of(ebase + m * SEG, 8)
            pltpu.make_async_copy(edges_hbm.at[pl.ds(off, SEG)],
                                  src_seg.at[sb], rsems[0]).wait()
            pltpu.make_async_copy(edges_hbm.at[pl.ds(off, SEG)],
                                  dst_seg.at[sb], rsems[1]).wait()

        refill(0)

        def zrow(r, carry):
            for k in range(DH // 16):
                robuf[r, pl.ds(k * 16, 16)] = jnp.zeros((16,), jnp.float32)
            return carry

        lax.fori_loop(0, RCHUNK, zrow, 0)
        for t in range((N_RCHUNK + NS - 1) // NS):
            j = s + t * NS

            @pl.when(j < N_RCHUNK)
            def _():
                r0 = pl.multiple_of(j * RCHUNK, 8)
                pltpu.sync_copy(robuf, agg.at[pl.ds(r0, RCHUNK)])
                sl = pl.ds(r0, RCHUNK)

                @pl.when(c == 0)
                def _():
                    pltpu.sync_copy(ha_hbm.at[sl], h_buf.at[sl])

                @pl.when(c == 1)
                def _():
                    pltpu.sync_copy(hb_hbm.at[sl], h_buf.at[sl])

        refill_wait(0)
        plsc.subcore_barrier()

        for m in range(NSEG):
            sb = m % 2
            if m + 1 < NSEG:
                refill(m + 1)

            def gather(j, buf):
                off = pl.multiple_of(j * C, 8)
                idx = src_seg.at[sb, pl.ds(off, C)]
                use_sp = lax.rem(j, 4) == 3

                @pl.when(use_sp)
                def _():
                    pltpu.async_copy(h_buf.at[idx], msg.at[buf], sems[buf])

                @pl.when(jnp.logical_not(use_sp) & (c == 0))
                def _():
                    pltpu.async_copy(ha_hbm.at[idx], msg.at[buf], sems[buf])

                @pl.when(jnp.logical_not(use_sp) & (c == 1))
                def _():
                    pltpu.async_copy(hb_hbm.at[idx], msg.at[buf], sems[buf])

            def wait(j, buf):
                off = pl.multiple_of(j * C, 8)
                pltpu.make_async_copy(h_buf.at[src_seg.at[sb, pl.ds(off, C)]],
                                      msg.at[buf], sems[buf]).wait()

            def scatter(j, buf):
                off = pl.multiple_of(j * C, 8)
                for t in range(C // 16):
                    dst_cur[pl.ds(t * 16, 16)] = (
                        dst_seg[sb, pl.ds(off + t * 16, 16)])
                pltpu.sync_copy(msg.at[buf], agg.at[dst_cur], add=True)

            for bb in range(NBUF - 1):
                gather(bb, bb)

            def group(g, carry):
                j0 = NBUF * g
                for bb in range(NBUF):
                    jn = j0 + bb + (NBUF - 1)

                    @pl.when(jn < CPS)
                    def _():
                        gather(jn, (bb + NBUF - 1) % NBUF)

                    wait(j0 + bb, bb)
                    scatter(j0 + bb, bb)
                return carry

            lax.fori_loop(0, CPS // NBUF, group, 0)
            if m + 1 < NSEG:
                refill_wait(m + 1)

        plsc.subcore_barrier()
        cb = pl.multiple_of(c * DH, 8)
        pltpu.sync_copy(b_hbm.at[pl.ds(cb, DH)], bvec)
        bias = [bvec[pl.ds(k * 16, 16)] for k in range(DH // 16)]
        lane = lax.iota(jnp.int32, 16)
        for t in range((N_RCHUNK + NS - 1) // NS):
            j = s + t * NS

            @pl.when(j < N_RCHUNK)
            def _():
                r0 = pl.multiple_of(j * RCHUNK, 8)
                pltpu.sync_copy(agg.at[pl.ds(r0, RCHUNK)], robuf)

                def brow(r, carry):
                    for k in range(DH // 16):
                        robuf[r, pl.ds(k * 16, 16)] = jnp.maximum(
                            robuf[r, pl.ds(k * 16, 16)] + bias[k], 0.0)
                    return carry

                lax.fori_loop(0, RCHUNK, brow, 0)
                for u in range(RCHUNK // 16):
                    ridx[pl.ds(u * 16, 16)] = (lane + (r0 + u * 16)) * 2 + c
                pltpu.sync_copy(robuf, out_hbm.at[ridx])

    return body(edges, ha, hb, b)


def kernel(inputs, adjacencies, W, b):
    ha, hb = _matmul(inputs, W)
    out2 = _sc_aggregate(adjacencies.reshape(-1), ha, hb, b)
    return out2.reshape(N_NODES, D)


# speedup vs baseline: 1.4153x; 1.4153x over previous
import functools

import jax
import jax.numpy as jnp
from jax import lax
from jax.experimental import pallas as pl
from jax.experimental.pallas import tpu as pltpu
from jax.experimental.pallas import tpu_sc as plsc

N_NODES = 10000
N_EDGES = 320000
D = 128
DH = D // 2

NC = 2
NS = 16
E_PER_T = N_EDGES // NS
C = 128
N_FULL = E_PER_T // C
C_TAIL = E_PER_T - N_FULL * C
NBUF = 4
RCHUNK = 80
N_RCHUNK = N_NODES // RCHUNK


def _mm_body(x_ref, w_ref, oa_ref, ob_ref):
    h = jnp.dot(x_ref[...], w_ref[...], preferred_element_type=jnp.float32)
    oa_ref[...] = h[:, :DH]
    ob_ref[...] = h[:, DH:]


def _matmul(x, W):
    return pl.pallas_call(
        _mm_body,
        grid=(5,),
        in_specs=[
            pl.BlockSpec((2000, D), lambda i: (i, 0)),
            pl.BlockSpec((D, D), lambda i: (0, 0)),
        ],
        out_specs=[
            pl.BlockSpec((2000, DH), lambda i: (i, 0)),
            pl.BlockSpec((2000, DH), lambda i: (i, 0)),
        ],
        out_shape=[
            jax.ShapeDtypeStruct((N_NODES, DH), jnp.float32),
            jax.ShapeDtypeStruct((N_NODES, DH), jnp.float32),
        ],
    )(x, W)


def _sc_aggregate(edges, ha, hb, b):
    mesh = plsc.VectorSubcoreMesh(core_axis_name="c", subcore_axis_name="s")

    @functools.partial(
        pl.kernel,
        out_type=jax.ShapeDtypeStruct((2 * N_NODES, DH), jnp.float32),
        mesh=mesh,
        compiler_params=pltpu.CompilerParams(use_tc_tiling_on_sc=False),
        scratch_types=[
            pltpu.VMEM((E_PER_T,), jnp.int32),
            pltpu.VMEM((E_PER_T,), jnp.int32),
            pltpu.VMEM((NBUF, C), jnp.int32),
            pltpu.VMEM((C_TAIL,), jnp.int32),
            pltpu.VMEM((NBUF, C, DH), jnp.float32),
            pltpu.VMEM((C_TAIL, DH), jnp.float32),
            pltpu.VMEM((RCHUNK, DH), jnp.float32),
            pltpu.VMEM((RCHUNK,), jnp.int32),
            pltpu.VMEM((DH,), jnp.float32),
            pltpu.VMEM_SHARED((N_NODES, DH), jnp.float32),
            [pltpu.SemaphoreType.DMA] * NBUF,
            [pltpu.SemaphoreType.DMA] * NBUF,
        ],
    )
    def body(edges_hbm, ha_hbm, hb_hbm, b_hbm, out_hbm, src_all, dst_all,
             dst_cur, dst_tail, msg, msg_tail, robuf, ridx, bvec, agg, sems,
             ssems):
        c = lax.axis_index("c")
        s = lax.axis_index("s")

        ebase = pl.multiple_of(s * E_PER_T, 8)
        pltpu.async_copy(edges_hbm.at[pl.ds(ebase, E_PER_T)], src_all, sems[0])
        pltpu.async_copy(edges_hbm.at[pl.ds(N_EDGES + ebase, E_PER_T)],
                         dst_all, sems[1])

        def zrow(r, carry):
            for k in range(DH // 16):
                robuf[r, pl.ds(k * 16, 16)] = jnp.zeros((16,), jnp.float32)
            return carry

        lax.fori_loop(0, RCHUNK, zrow, 0)
        for t in range((N_RCHUNK + NS - 1) // NS):
            j = s + t * NS

            @pl.when(j < N_RCHUNK)
            def _():
                r0 = pl.multiple_of(j * RCHUNK, 8)
                pltpu.sync_copy(robuf, agg.at[pl.ds(r0, RCHUNK)])

        pltpu.make_async_copy(edges_hbm.at[pl.ds(ebase, E_PER_T)],
                              src_all, sems[0]).wait()
        pltpu.make_async_copy(edges_hbm.at[pl.ds(ebase, E_PER_T)],
                              dst_all, sems[1]).wait()
        plsc.subcore_barrier()

        def gather(j, buf):
            off = pl.multiple_of(j * C, 8)
            idx = src_all.at[pl.ds(off, C)]

            @pl.when(c == 0)
            def _():
                pltpu.async_copy(ha_hbm.at[idx], msg.at[buf], sems[buf])

            @pl.when(c == 1)
            def _():
                pltpu.async_copy(hb_hbm.at[idx], msg.at[buf], sems[buf])

        def wait(j, buf):
            off = pl.multiple_of(j * C, 8)
            pltpu.make_async_copy(ha_hbm.at[src_all.at[pl.ds(off, C)]],
                                  msg.at[buf], sems[buf]).wait()

        def scatter(j, buf):
            off = pl.multiple_of(j * C, 8)
            for t in range(C // 16):
                dst_cur[buf, pl.ds(t * 16, 16)] = (
                    dst_all[pl.ds(off + t * 16, 16)])
            pltpu.async_copy(msg.at[buf], agg.at[dst_cur.at[buf]],
                             ssems[buf], add=True)

        def scatter_drain(buf):
            pltpu.make_async_copy(msg.at[buf], agg.at[dst_cur.at[buf]],
                                  ssems[buf]).wait()

        for b in range(NBUF - 1):
            gather(b, b)

        def group(g, carry):
            j0 = NBUF * g
            for b in range(NBUF):
                jn = j0 + b + (NBUF - 1)
                sn = (b + NBUF - 1) % NBUF

                @pl.when(jn < N_FULL)
                def _():
                    @pl.when(jn >= NBUF)
                    def _():
                        scatter_drain(sn)

                    gather(jn, sn)

                wait(j0 + b, b)
                scatter(j0 + b, b)
            return carry

        lax.fori_loop(0, N_FULL // NBUF, group, 0)
        for b in range(NBUF):
            scatter_drain(b)

        toff = pl.multiple_of(N_FULL * C, 8)
        tidx = src_all.at[pl.ds(toff, C_TAIL)]

        @pl.when(c == 0)
        def _():
            pltpu.async_copy(ha_hbm.at[tidx], msg_tail, sems[0])

        @pl.when(c == 1)
        def _():
            pltpu.async_copy(hb_hbm.at[tidx], msg_tail, sems[0])

        pltpu.make_async_copy(ha_hbm.at[tidx], msg_tail, sems[0]).wait()
        for t in range(C_TAIL // 16):
            dst_tail[pl.ds(t * 16, 16)] = dst_all[pl.ds(toff + t * 16, 16)]
        pltpu.sync_copy(msg_tail, agg.at[dst_tail], add=True)

        plsc.subcore_barrier()
        cb = pl.multiple_of(c * DH, 8)
        pltpu.sync_copy(b_hbm.at[pl.ds(cb, DH)], bvec)
        bias = [bvec[pl.ds(k * 16, 16)] for k in range(DH // 16)]
        lane = lax.iota(jnp.int32, 16)
        for t in range((N_RCHUNK + NS - 1) // NS):
            j = s + t * NS

            @pl.when(j < N_RCHUNK)
            def _():
                r0 = pl.multiple_of(j * RCHUNK, 8)
                pltpu.sync_copy(agg.at[pl.ds(r0, RCHUNK)], robuf)

                def brow(r, carry):
                    for k in range(DH // 16):
                        robuf[r, pl.ds(k * 16, 16)] = jnp.maximum(
                            robuf[r, pl.ds(k * 16, 16)] + bias[k], 0.0)
                    return carry

                lax.fori_loop(0, RCHUNK, brow, 0)
                for u in range(RCHUNK // 16):
                    ridx[pl.ds(u * 16, 16)] = (lane + (r0 + u * 16)) * 2 + c
                pltpu.sync_copy(robuf, out_hbm.at[ridx])

    return body(edges, ha, hb, b)


def kernel(inputs, adjacencies, W, b):
    ha, hb = _matmul(inputs, W)
    out2 = _sc_aggregate(adjacencies.reshape(-1), ha, hb, b)
    return out2.reshape(N_NODES, D)
